# baseline (device time: 171860 ns/iter reference)
import jax
import jax.numpy as jnp
from jax import lax
from jax.experimental import pallas as pl
from jax.experimental.pallas import tpu as pltpu

N_DEV = 4
M = 4096
N = 2048
M_CHUNK = M // N_DEV
COLS = N // 2
WAVES = 4
MW = M_CHUNK // WAVES
N_SPLIT = 1
M_TILE = MW // N_SPLIT


def kernel(x, w_mat):
    m, k_shard = x.shape
    _, n = w_mat.shape
    assert (m, n) == (M, N)
    xb = x.astype(jnp.bfloat16)
    wb = w_mat.astype(jnp.bfloat16)

    def body(x_ref, w_ref, out_ref, seed_a, seed_b, p_a, p_b, rs_a, rs_b,
             send_a, recv_a, send_b, recv_b):
        my = lax.axis_index("i")
        left = (my + N_DEV - 1) % N_DEV
        right = (my + 1) % N_DEV

        barrier_sem = pltpu.get_barrier_semaphore()
        for nbr in [left, right]:
            pl.semaphore_signal(barrier_sem, inc=1, device_id=(nbr,),
                                device_id_type=pl.DeviceIdType.MESH)
        pl.semaphore_wait(barrier_sem, 2)

        def row0_r(r, w):
            return ((my + N_DEV - r) % N_DEV) * M_CHUNK + w * MW

        def row0_l(r, w):
            return ((my + r) % N_DEV) * M_CHUNK + w * MW

        def fill_partial(dst3, w, row0, col0):
            for s in range(N_SPLIT):
                dst3[w, pl.ds(s * M_TILE, M_TILE), :] = lax.dot_general(
                    x_ref[pl.ds(row0 + s * M_TILE, M_TILE), :],
                    w_ref[:, pl.ds(col0, COLS)], (((1,), (0,)), ((), ())),
                    preferred_element_type=jnp.float32,
                ).astype(jnp.bfloat16)

        def start_rs(w, h, src_a, src_b, dst_a, dst_b):
            i = WAVES * h + w
            ra = pltpu.make_async_remote_copy(
                src_ref=src_a, dst_ref=dst_a,
                send_sem=send_a.at[i], recv_sem=recv_a.at[i],
                device_id=(right,), device_id_type=pl.DeviceIdType.MESH,
            )
            rb = pltpu.make_async_remote_copy(
                src_ref=src_b, dst_ref=dst_b,
                send_sem=send_b.at[i], recv_sem=recv_b.at[i],
                device_id=(left,), device_id_type=pl.DeviceIdType.MESH,
            )
            ra.start()
            rb.start()
            return ra, rb

        def ag_buf(w, g):
            bufs_a = [seed_a.at[w], rs_a.at[w, 0], rs_a.at[w, 1],
                      seed_a.at[w]]
            bufs_b = [seed_b.at[w], rs_b.at[w, 0], rs_b.at[w, 1],
                      seed_b.at[w]]
            return bufs_a[g], bufs_b[g]

        def start_ag(w, g):
            i = WAVES * (N_DEV - 1 + g) + w
            src_a_, src_b_ = ag_buf(w, g)
            dst_a_, dst_b_ = ag_buf(w, g + 1)
            ra = pltpu.make_async_remote_copy(
                src_ref=src_a_, dst_ref=dst_a_,
                send_sem=send_a.at[i], recv_sem=recv_a.at[i],
                device_id=(right,), device_id_type=pl.DeviceIdType.MESH,
            )
            rb = pltpu.make_async_remote_copy(
                src_ref=src_b_, dst_ref=dst_b_,
                send_sem=send_b.at[i], recv_sem=recv_b.at[i],
                device_id=(left,), device_id_type=pl.DeviceIdType.MESH,
            )
            ra.start()
            rb.start()
            return ra, rb

        cur = [None] * WAVES
        for w in range(WAVES):
            fill_partial(seed_a, w, row0_r(0, w), 0)
            fill_partial(seed_b, w, row0_l(0, w), COLS)
            cur[w] = start_rs(w, 0, seed_a.at[w], seed_b.at[w],
                              rs_a.at[w, 0], rs_b.at[w, 0])

        for h in range(N_DEV - 1):
            for w in range(WAVES):
                dst_a = rs_a.at[w, h] if h < N_DEV - 2 else seed_a.at[w]
                dst_b = rs_b.at[w, h] if h < N_DEV - 2 else seed_b.at[w]
                fill_partial(p_a, w, row0_r(h + 1, w), 0)
                fill_partial(p_b, w, row0_l(h + 1, w), COLS)
                ra, rb = cur[w]
                ra.wait()
                rb.wait()
                for s in range(N_SPLIT):
                    sl = pl.ds(s * M_TILE, M_TILE)
                    acc_a = (dst_a[sl, :].astype(jnp.float32)
                             + p_a[w, sl, :].astype(jnp.float32))
                    acc_b = (dst_b[sl, :].astype(jnp.float32)
                             + p_b[w, sl, :].astype(jnp.float32))
                    if h < N_DEV - 2:
                        dst_a[sl, :] = acc_a.astype(jnp.bfloat16)
                        dst_b[sl, :] = acc_b.astype(jnp.bfloat16)
                    else:
                        val_a = jnp.maximum(acc_a, 0.0).astype(jnp.bfloat16)
                        val_b = jnp.maximum(acc_b, 0.0).astype(jnp.bfloat16)
                        out_ref[pl.ds(row0_r(N_DEV - 1, w) + s * M_TILE,
                                      M_TILE), pl.ds(0, COLS)] = val_a
                        out_ref[pl.ds(row0_l(N_DEV - 1, w) + s * M_TILE,
                                      M_TILE), pl.ds(COLS, COLS)] = val_b
                        seed_a[w, sl, :] = val_a
                        seed_b[w, sl, :] = val_b
                if h < N_DEV - 2:
                    cur[w] = start_rs(w, h + 1, dst_a, dst_b,
                                      rs_a.at[w, h + 1] if h + 1 < N_DEV - 2
                                      else seed_a.at[w],
                                      rs_b.at[w, h + 1] if h + 1 < N_DEV - 2
                                      else seed_b.at[w])
                else:
                    cur[w] = start_ag(w, 0)

        for g in range(N_DEV - 1):
            for w in range(WAVES):
                ra, rb = cur[w]
                ra.wait()
                rb.wait()
                if g < N_DEV - 2:
                    cur[w] = start_ag(w, g + 1)
                buf_a, buf_b = ag_buf(w, g + 1)
                row_a = ((my + N_DEV - g) % N_DEV) * M_CHUNK + w * MW
                row_b = ((my + g) % N_DEV) * M_CHUNK + w * MW
                out_ref[pl.ds(row_a, MW), pl.ds(0, COLS)] = buf_a[:, :]
                out_ref[pl.ds(row_b, MW), pl.ds(COLS, COLS)] = buf_b[:, :]

    return pl.pallas_call(
        body,
        out_shape=jax.ShapeDtypeStruct((M, N), jnp.bfloat16),
        in_specs=[
            pl.BlockSpec(memory_space=pltpu.VMEM),
            pl.BlockSpec(memory_space=pltpu.VMEM),
        ],
        out_specs=pl.BlockSpec(memory_space=pltpu.VMEM),
        scratch_shapes=[
            pltpu.VMEM((WAVES, MW, COLS), jnp.bfloat16),
            pltpu.VMEM((WAVES, MW, COLS), jnp.bfloat16),
            pltpu.VMEM((WAVES, MW, COLS), jnp.bfloat16),
            pltpu.VMEM((WAVES, MW, COLS), jnp.bfloat16),
            pltpu.VMEM((WAVES, 2, MW, COLS), jnp.bfloat16),
            pltpu.VMEM((WAVES, 2, MW, COLS), jnp.bfloat16),
            pltpu.SemaphoreType.DMA((WAVES * 6,)),
            pltpu.SemaphoreType.DMA((WAVES * 6,)),
            pltpu.SemaphoreType.DMA((WAVES * 6,)),
            pltpu.SemaphoreType.DMA((WAVES * 6,)),
        ],
        compiler_params=pltpu.CompilerParams(
            collective_id=0, vmem_limit_bytes=34 * 1024 * 1024),
    )(xb, wb)


# device time: 167334 ns/iter; 1.0270x vs baseline; 1.0270x over previous
import jax
import jax.numpy as jnp
from jax import lax
from jax.experimental import pallas as pl
from jax.experimental.pallas import tpu as pltpu

N_DEV = 4
M = 4096
N = 2048
M_CHUNK = M // N_DEV
COLS = N // 2
WAVES = 4
MW = M_CHUNK // WAVES
N_SPLIT = 1
M_TILE = MW // N_SPLIT


def kernel(x, w_mat):
    m, k_shard = x.shape
    _, n = w_mat.shape
    assert (m, n) == (M, N)
    xb = x.astype(jnp.bfloat16)
    wb = w_mat.astype(jnp.bfloat16)

    def body(x_ref, w_ref, out_ref, seed_a, seed_b, p_a, p_b, rs_a, rs_b,
             send_a, recv_a, send_b, recv_b, copy_sems):
        out_copies = []

        def copy_out(buf, row0, col0, idx):
            c = pltpu.make_async_copy(
                buf, out_ref.at[pl.ds(row0, MW), pl.ds(col0, COLS)],
                copy_sems.at[idx])
            c.start()
            out_copies.append(c)
        my = lax.axis_index("i")
        left = (my + N_DEV - 1) % N_DEV
        right = (my + 1) % N_DEV

        barrier_sem = pltpu.get_barrier_semaphore()
        for nbr in [left, right]:
            pl.semaphore_signal(barrier_sem, inc=1, device_id=(nbr,),
                                device_id_type=pl.DeviceIdType.MESH)
        pl.semaphore_wait(barrier_sem, 2)

        def row0_r(r, w):
            return ((my + N_DEV - r) % N_DEV) * M_CHUNK + w * MW

        def row0_l(r, w):
            return ((my + r) % N_DEV) * M_CHUNK + w * MW

        def fill_partial(dst3, w, row0, col0):
            for s in range(N_SPLIT):
                dst3[w, pl.ds(s * M_TILE, M_TILE), :] = lax.dot_general(
                    x_ref[pl.ds(row0 + s * M_TILE, M_TILE), :],
                    w_ref[:, pl.ds(col0, COLS)], (((1,), (0,)), ((), ())),
                    preferred_element_type=jnp.float32,
                ).astype(jnp.bfloat16)

        def start_rs(w, h, src_a, src_b, dst_a, dst_b):
            i = WAVES * h + w
            ra = pltpu.make_async_remote_copy(
                src_ref=src_a, dst_ref=dst_a,
                send_sem=send_a.at[i], recv_sem=recv_a.at[i],
                device_id=(right,), device_id_type=pl.DeviceIdType.MESH,
            )
            rb = pltpu.make_async_remote_copy(
                src_ref=src_b, dst_ref=dst_b,
                send_sem=send_b.at[i], recv_sem=recv_b.at[i],
                device_id=(left,), device_id_type=pl.DeviceIdType.MESH,
            )
            ra.start()
            rb.start()
            return ra, rb

        def ag_buf(w, g):
            bufs_a = [seed_a.at[w], rs_a.at[w, 0], rs_a.at[w, 1],
                      p_a.at[w]]
            bufs_b = [seed_b.at[w], rs_b.at[w, 0], rs_b.at[w, 1],
                      p_b.at[w]]
            return bufs_a[g], bufs_b[g]

        def start_ag(w, g):
            i = WAVES * (N_DEV - 1 + g) + w
            src_a_, src_b_ = ag_buf(w, g)
            dst_a_, dst_b_ = ag_buf(w, g + 1)
            ra = pltpu.make_async_remote_copy(
                src_ref=src_a_, dst_ref=dst_a_,
                send_sem=send_a.at[i], recv_sem=recv_a.at[i],
                device_id=(right,), device_id_type=pl.DeviceIdType.MESH,
            )
            rb = pltpu.make_async_remote_copy(
                src_ref=src_b_, dst_ref=dst_b_,
                send_sem=send_b.at[i], recv_sem=recv_b.at[i],
                device_id=(left,), device_id_type=pl.DeviceIdType.MESH,
            )
            ra.start()
            rb.start()
            return ra, rb

        cur = [None] * WAVES
        for w in range(WAVES):
            fill_partial(seed_a, w, row0_r(0, w), 0)
            fill_partial(seed_b, w, row0_l(0, w), COLS)
            cur[w] = start_rs(w, 0, seed_a.at[w], seed_b.at[w],
                              rs_a.at[w, 0], rs_b.at[w, 0])

        for h in range(N_DEV - 1):
            for w in range(WAVES):
                dst_a = rs_a.at[w, h] if h < N_DEV - 2 else seed_a.at[w]
                dst_b = rs_b.at[w, h] if h < N_DEV - 2 else seed_b.at[w]
                fill_partial(p_a, w, row0_r(h + 1, w), 0)
                fill_partial(p_b, w, row0_l(h + 1, w), COLS)
                ra, rb = cur[w]
                ra.wait()
                rb.wait()
                for s in range(N_SPLIT):
                    sl = pl.ds(s * M_TILE, M_TILE)
                    acc_a = (dst_a[sl, :].astype(jnp.float32)
                             + p_a[w, sl, :].astype(jnp.float32))
                    acc_b = (dst_b[sl, :].astype(jnp.float32)
                             + p_b[w, sl, :].astype(jnp.float32))
                    if h < N_DEV - 2:
                        dst_a[sl, :] = acc_a.astype(jnp.bfloat16)
                        dst_b[sl, :] = acc_b.astype(jnp.bfloat16)
                    else:
                        seed_a[w, sl, :] = (
                            jnp.maximum(acc_a, 0.0).astype(jnp.bfloat16))
                        seed_b[w, sl, :] = (
                            jnp.maximum(acc_b, 0.0).astype(jnp.bfloat16))
                if h < N_DEV - 2:
                    cur[w] = start_rs(w, h + 1, dst_a, dst_b,
                                      rs_a.at[w, h + 1] if h + 1 < N_DEV - 2
                                      else seed_a.at[w],
                                      rs_b.at[w, h + 1] if h + 1 < N_DEV - 2
                                      else seed_b.at[w])
                else:
                    cur[w] = start_ag(w, 0)
                    copy_out(seed_a.at[w], row0_r(N_DEV - 1, w), 0,
                             w * 8 + 0)
                    copy_out(seed_b.at[w], row0_l(N_DEV - 1, w), COLS,
                             w * 8 + 1)

        for g in range(N_DEV - 1):
            for w in range(WAVES):
                ra, rb = cur[w]
                ra.wait()
                rb.wait()
                if g < N_DEV - 2:
                    cur[w] = start_ag(w, g + 1)
                buf_a, buf_b = ag_buf(w, g + 1)
                row_a = ((my + N_DEV - g) % N_DEV) * M_CHUNK + w * MW
                row_b = ((my + g) % N_DEV) * M_CHUNK + w * MW
                copy_out(buf_a, row_a, 0, w * 8 + 2 + 2 * g)
                copy_out(buf_b, row_b, COLS, w * 8 + 3 + 2 * g)

        for c in out_copies:
            c.wait()

    return pl.pallas_call(
        body,
        out_shape=jax.ShapeDtypeStruct((M, N), jnp.bfloat16),
        in_specs=[
            pl.BlockSpec(memory_space=pltpu.VMEM),
            pl.BlockSpec(memory_space=pltpu.VMEM),
        ],
        out_specs=pl.BlockSpec(memory_space=pl.ANY),
        scratch_shapes=[
            pltpu.VMEM((WAVES, MW, COLS), jnp.bfloat16),
            pltpu.VMEM((WAVES, MW, COLS), jnp.bfloat16),
            pltpu.VMEM((WAVES, MW, COLS), jnp.bfloat16),
            pltpu.VMEM((WAVES, MW, COLS), jnp.bfloat16),
            pltpu.VMEM((WAVES, 2, MW, COLS), jnp.bfloat16),
            pltpu.VMEM((WAVES, 2, MW, COLS), jnp.bfloat16),
            pltpu.SemaphoreType.DMA((WAVES * 6,)),
            pltpu.SemaphoreType.DMA((WAVES * 6,)),
            pltpu.SemaphoreType.DMA((WAVES * 6,)),
            pltpu.SemaphoreType.DMA((WAVES * 6,)),
            pltpu.SemaphoreType.DMA((WAVES * 8,)),
        ],
        compiler_params=pltpu.CompilerParams(
            collective_id=0, vmem_limit_bytes=34 * 1024 * 1024),
    )(xb, wb)


# device time: 164959 ns/iter; 1.0418x vs baseline; 1.0144x over previous
import jax
import jax.numpy as jnp
from jax import lax
from jax.experimental import pallas as pl
from jax.experimental.pallas import tpu as pltpu

N_DEV = 4
M = 4096
N = 2048
M_CHUNK = M // N_DEV
COLS = N // 2
WAVES = 4
MW = M_CHUNK // WAVES
N_SPLIT = 1
M_TILE = MW // N_SPLIT


def kernel(x, w_mat):
    m, k_shard = x.shape
    _, n = w_mat.shape
    assert (m, n) == (M, N)

    def body(x_ref, w_ref, out_ref, wb_ref, seed_a, seed_b, p_a, p_b,
             rs_a, rs_b, send_a, recv_a, send_b, recv_b, copy_sems):
        out_copies = []

        def copy_out(buf, row0, col0, idx):
            c = pltpu.make_async_copy(
                buf, out_ref.at[pl.ds(row0, MW), pl.ds(col0, COLS)],
                copy_sems.at[idx])
            c.start()
            out_copies.append(c)
        my = lax.axis_index("i")
        left = (my + N_DEV - 1) % N_DEV
        right = (my + 1) % N_DEV

        barrier_sem = pltpu.get_barrier_semaphore()
        for nbr in [left, right]:
            pl.semaphore_signal(barrier_sem, inc=1, device_id=(nbr,),
                                device_id_type=pl.DeviceIdType.MESH)
        pl.semaphore_wait(barrier_sem, 2)

        def row0_r(r, w):
            return ((my + N_DEV - r) % N_DEV) * M_CHUNK + w * MW

        def row0_l(r, w):
            return ((my + r) % N_DEV) * M_CHUNK + w * MW

        for cc in range(2):
            wb_ref[:, pl.ds(cc * COLS, COLS)] = (
                w_ref[:, pl.ds(cc * COLS, COLS)].astype(jnp.bfloat16))

        def fill_partial(dst3, w, row0, col0):
            for s in range(N_SPLIT):
                dst3[w, pl.ds(s * M_TILE, M_TILE), :] = lax.dot_general(
                    x_ref[pl.ds(row0 + s * M_TILE, M_TILE), :].astype(
                        jnp.bfloat16),
                    wb_ref[:, pl.ds(col0, COLS)], (((1,), (0,)), ((), ())),
                    preferred_element_type=jnp.float32,
                ).astype(jnp.bfloat16)

        def start_rs(w, h, src_a, src_b, dst_a, dst_b):
            i = WAVES * h + w
            ra = pltpu.make_async_remote_copy(
                src_ref=src_a, dst_ref=dst_a,
                send_sem=send_a.at[i], recv_sem=recv_a.at[i],
                device_id=(right,), device_id_type=pl.DeviceIdType.MESH,
            )
            rb = pltpu.make_async_remote_copy(
                src_ref=src_b, dst_ref=dst_b,
                send_sem=send_b.at[i], recv_sem=recv_b.at[i],
                device_id=(left,), device_id_type=pl.DeviceIdType.MESH,
            )
            ra.start()
            rb.start()
            return ra, rb

        def ag_buf(w, g):
            bufs_a = [seed_a.at[w], rs_a.at[w, 0], rs_a.at[w, 1],
                      p_a.at[w]]
            bufs_b = [seed_b.at[w], rs_b.at[w, 0], rs_b.at[w, 1],
                      p_b.at[w]]
            return bufs_a[g], bufs_b[g]

        def start_ag(w, g):
            i = WAVES * (N_DEV - 1 + g) + w
            src_a_, src_b_ = ag_buf(w, g)
            dst_a_, dst_b_ = ag_buf(w, g + 1)
            ra = pltpu.make_async_remote_copy(
                src_ref=src_a_, dst_ref=dst_a_,
                send_sem=send_a.at[i], recv_sem=recv_a.at[i],
                device_id=(right,), device_id_type=pl.DeviceIdType.MESH,
            )
            rb = pltpu.make_async_remote_copy(
                src_ref=src_b_, dst_ref=dst_b_,
                send_sem=send_b.at[i], recv_sem=recv_b.at[i],
                device_id=(left,), device_id_type=pl.DeviceIdType.MESH,
            )
            ra.start()
            rb.start()
            return ra, rb

        cur = [None] * WAVES
        for w in range(WAVES):
            fill_partial(seed_a, w, row0_r(0, w), 0)
            fill_partial(seed_b, w, row0_l(0, w), COLS)
            cur[w] = start_rs(w, 0, seed_a.at[w], seed_b.at[w],
                              rs_a.at[w, 0], rs_b.at[w, 0])

        for h in range(N_DEV - 1):
            for w in range(WAVES):
                dst_a = rs_a.at[w, h] if h < N_DEV - 2 else seed_a.at[w]
                dst_b = rs_b.at[w, h] if h < N_DEV - 2 else seed_b.at[w]
                fill_partial(p_a, w, row0_r(h + 1, w), 0)
                fill_partial(p_b, w, row0_l(h + 1, w), COLS)
                ra, rb = cur[w]
                ra.wait()
                rb.wait()
                for s in range(N_SPLIT):
                    sl = pl.ds(s * M_TILE, M_TILE)
                    acc_a = (dst_a[sl, :].astype(jnp.float32)
                             + p_a[w, sl, :].astype(jnp.float32))
                    acc_b = (dst_b[sl, :].astype(jnp.float32)
                             + p_b[w, sl, :].astype(jnp.float32))
                    if h < N_DEV - 2:
                        dst_a[sl, :] = acc_a.astype(jnp.bfloat16)
                        dst_b[sl, :] = acc_b.astype(jnp.bfloat16)
                    else:
                        seed_a[w, sl, :] = (
                            jnp.maximum(acc_a, 0.0).astype(jnp.bfloat16))
                        seed_b[w, sl, :] = (
                            jnp.maximum(acc_b, 0.0).astype(jnp.bfloat16))
                if h < N_DEV - 2:
                    cur[w] = start_rs(w, h + 1, dst_a, dst_b,
                                      rs_a.at[w, h + 1] if h + 1 < N_DEV - 2
                                      else seed_a.at[w],
                                      rs_b.at[w, h + 1] if h + 1 < N_DEV - 2
                                      else seed_b.at[w])
                else:
                    cur[w] = start_ag(w, 0)
                    copy_out(seed_a.at[w], row0_r(N_DEV - 1, w), 0,
                             w * 8 + 0)
                    copy_out(seed_b.at[w], row0_l(N_DEV - 1, w), COLS,
                             w * 8 + 1)

        for g in range(N_DEV - 1):
            for w in range(WAVES):
                ra, rb = cur[w]
                ra.wait()
                rb.wait()
                if g < N_DEV - 2:
                    cur[w] = start_ag(w, g + 1)
                buf_a, buf_b = ag_buf(w, g + 1)
                row_a = ((my + N_DEV - g) % N_DEV) * M_CHUNK + w * MW
                row_b = ((my + g) % N_DEV) * M_CHUNK + w * MW
                copy_out(buf_a, row_a, 0, w * 8 + 2 + 2 * g)
                copy_out(buf_b, row_b, COLS, w * 8 + 3 + 2 * g)

        for c in out_copies:
            c.wait()

    return pl.pallas_call(
        body,
        out_shape=jax.ShapeDtypeStruct((M, N), jnp.bfloat16),
        in_specs=[
            pl.BlockSpec(memory_space=pltpu.VMEM),
            pl.BlockSpec(memory_space=pltpu.VMEM),
        ],
        out_specs=pl.BlockSpec(memory_space=pl.ANY),
        scratch_shapes=[
            pltpu.VMEM((1024, N), jnp.bfloat16),
            pltpu.VMEM((WAVES, MW, COLS), jnp.bfloat16),
            pltpu.VMEM((WAVES, MW, COLS), jnp.bfloat16),
            pltpu.VMEM((WAVES, MW, COLS), jnp.bfloat16),
            pltpu.VMEM((WAVES, MW, COLS), jnp.bfloat16),
            pltpu.VMEM((WAVES, 2, MW, COLS), jnp.bfloat16),
            pltpu.VMEM((WAVES, 2, MW, COLS), jnp.bfloat16),
            pltpu.SemaphoreType.DMA((WAVES * 6,)),
            pltpu.SemaphoreType.DMA((WAVES * 6,)),
            pltpu.SemaphoreType.DMA((WAVES * 6,)),
            pltpu.SemaphoreType.DMA((WAVES * 6,)),
            pltpu.SemaphoreType.DMA((WAVES * 8,)),
        ],
        compiler_params=pltpu.CompilerParams(
            collective_id=0, vmem_limit_bytes=38 * 1024 * 1024),
    )(x, w_mat)
